# baseline (speedup 1.0000x reference)
import functools

import jax
import jax.numpy as jnp
from jax.experimental import pallas as pl
from jax.experimental.pallas import tpu as pltpu

_FCLAMP = 30.0
_CH = 128


def _proj_body(nblocks, x_ref, w_ref, a_ref, h_ref, f_ref, hs_ref, acc_ref):
    i = pl.program_id(0)
    h = jnp.dot(x_ref[...], w_ref[...], preferred_element_type=jnp.float32)
    h_ref[...] = h
    f_ref[...] = jnp.minimum(
        jnp.dot(h, a_ref[...], preferred_element_type=jnp.float32), _FCLAMP)

    @pl.when(i == 0)
    def _init():
        acc_ref[...] = jnp.zeros_like(acc_ref)

    acc_ref[...] += jnp.sum(h, axis=0, keepdims=True)
    @pl.when(i == nblocks - 1)
    def _emit():
        hs_ref[...] = acc_ref[...]


def _proj(x, W, a_src, block_rows):
    N, d_in = x.shape
    d_out = W.shape[1]
    nblocks = N // block_rows
    return pl.pallas_call(
        functools.partial(_proj_body, nblocks),
        grid=(nblocks,),
        in_specs=[
            pl.BlockSpec((block_rows, d_in), lambda i: (i, 0)),
            pl.BlockSpec((d_in, d_out), lambda i: (0, 0)),
            pl.BlockSpec((d_out, 1), lambda i: (0, 0)),
        ],
        out_specs=[
            pl.BlockSpec((block_rows, d_out), lambda i: (i, 0)),
            pl.BlockSpec((block_rows, 1), lambda i: (i, 0)),
            pl.BlockSpec((1, d_out), lambda i: (0, 0)),
        ],
        out_shape=[
            jax.ShapeDtypeStruct((N, d_out), jnp.float32),
            jax.ShapeDtypeStruct((N, 1), jnp.float32),
            jax.ShapeDtypeStruct((1, d_out), jnp.float32),
        ],
        scratch_shapes=[pltpu.VMEM((1, d_out), jnp.float32)],
    )(x, W, a_src.reshape(d_out, 1))


def _attend(fs, a_dst_row, h_ref, adjf_chunks):
    acc = None
    l = None
    for c, adjf in adjf_chunks:
        h_c = h_ref[pl.ds(c * _CH, _CH), :]
        fd = jnp.minimum(
            jax.lax.dot_general(a_dst_row, h_c,
                                dimension_numbers=(((1,), (1,)), ((), ())),
                                preferred_element_type=jnp.float32),
            _FCLAMP)
        s = fs + fd
        e = jnp.maximum(s, 0.2 * s)
        p = jnp.exp(e) * adjf
        ones = jnp.ones((_CH, 1), jnp.float32)
        pa = jnp.dot(p, h_c, preferred_element_type=jnp.float32)
        pl_ = jnp.dot(p, ones, preferred_element_type=jnp.float32)
        acc = pa if acc is None else acc + pa
        l = pl_ if l is None else l + pl_
    return acc, l


def _finish(n, out_ref, l_ref, acc_ref, hs_ref):
    l = l_ref[...]
    z = jnp.where(l > 0, acc_ref[...] / l, hs_ref[...] / n)
    out_ref[...] = jnp.where(z > 0, z, jnp.exp(z) - 1.0)


def _flash1_body(n, bc, n_col_blocks,
                 f_src_ref, h_ref, a_dst_ref, hsum_ref, adj_ref,
                 out_ref, mask_ref, l_ref, acc_ref):
    j = pl.program_id(1)
    last = n_col_blocks - 1
    fs = f_src_ref[...]
    a_row = a_dst_ref[...]

    @pl.when(j == 0)
    def _init():
        l_ref[...] = jnp.zeros_like(l_ref)
        acc_ref[...] = jnp.zeros_like(acc_ref)

    @pl.when(j < last)
    def _inner():
        def chunks():
            for c in range(bc // _CH):
                adj_c = adj_ref[:, c * _CH:(c + 1) * _CH]
                mask_ref[:, c * _CH:(c + 1) * _CH] = adj_c.astype(jnp.int8)
                yield c, adj_c.astype(jnp.float32)
        acc, l = _attend(fs, a_row, h_ref, chunks(), bc)
        acc_ref[...] += acc
        l_ref[...] += l

    @pl.when(j == last)
    def _tail():
        def chunks():
            for c in range(bc // _CH):
                rem = n - last * bc - c * _CH
                adj_c = adj_ref[:, c * _CH:(c + 1) * _CH]
                ok = jax.lax.broadcasted_iota(
                    jnp.int32, adj_c.shape, 1) < rem
                adj_m = adj_c * ok.astype(jnp.int32)
                mask_ref[:, c * _CH:(c + 1) * _CH] = adj_m.astype(jnp.int8)
                yield c, adj_m.astype(jnp.float32)
        acc, l = _attend(fs, a_row, h_ref, chunks(), bc)
        acc_ref[...] += acc
        l_ref[...] += l
        _finish(n, out_ref, l_ref, acc_ref, hsum_ref)


def _flash2_body(n, bc, n_col_blocks,
                 f_src_ref, h_ref, a_dst_ref, hsum_ref, mask_ref,
                 out_ref, l_ref, acc_ref):
    j = pl.program_id(1)
    fs = f_src_ref[...]
    a_row = a_dst_ref[...]

    @pl.when(j == 0)
    def _init():
        l_ref[...] = jnp.zeros_like(l_ref)
        acc_ref[...] = jnp.zeros_like(acc_ref)

    def chunks():
        for c in range(bc // _CH):
            yield c, mask_ref[:, c * _CH:(c + 1) * _CH].astype(jnp.float32)
    acc, l = _attend(fs, a_row, h_ref, chunks())
    acc_ref[...] += acc
    l_ref[...] += l

    @pl.when(j == n_col_blocks - 1)
    def _tail():
        _finish(n, out_ref, l_ref, acc_ref, hsum_ref)


def _flash_layer(f_src, h_pad, a_dst, hsum, adj, emit_mask,
                 block_rows, block_cols):
    N = f_src.shape[0]
    D = h_pad.shape[1]
    n_pad = h_pad.shape[0]
    nr = pl.cdiv(N, block_rows)
    nc = n_pad // block_cols
    out_shape = [jax.ShapeDtypeStruct((N, D), jnp.float32)]
    out_specs = [pl.BlockSpec((block_rows, D), lambda i, j: (i, 0))]
    if emit_mask:
        body = functools.partial(_flash1_body, N, block_cols, nc)
        out_shape.append(jax.ShapeDtypeStruct((N, n_pad), jnp.int8))
        out_specs.append(pl.BlockSpec((block_rows, block_cols),
                                      lambda i, j: (i, j)))
    else:
        body = functools.partial(_flash2_body, N, block_cols, nc)
    res = pl.pallas_call(
        body,
        grid=(nr, nc),
        in_specs=[
            pl.BlockSpec((block_rows, 1), lambda i, j: (i, 0)),
            pl.BlockSpec((block_cols, D), lambda i, j: (j, 0)),
            pl.BlockSpec((1, D), lambda i, j: (0, 0)),
            pl.BlockSpec((1, D), lambda i, j: (0, 0)),
            pl.BlockSpec((block_rows, block_cols), lambda i, j: (i, j)),
        ],
        out_specs=out_specs,
        out_shape=out_shape,
        scratch_shapes=[
            pltpu.VMEM((block_rows, 1), jnp.float32),
            pltpu.VMEM((block_rows, D), jnp.float32),
        ],
    )(f_src, h_pad, a_dst.reshape(1, D), hsum, adj)
    return res if emit_mask else res[0]


def kernel(x, adj, W_e0, a_src_e0, a_dst_e0, W_d0, a_src_d0, a_dst_d0):
    N = x.shape[0]
    BC = 1024
    n_pad = ((N + BC - 1) // BC) * BC
    br = 2000 if N % 2000 == 0 else N
    h1, f1, hs1 = _proj(x, W_e0, a_src_e0, block_rows=br)
    h1p = jnp.pad(h1, ((0, n_pad - N), (0, 0)))
    h_enc, mask8 = _flash_layer(f1, h1p, a_dst_e0, hs1, adj,
                                emit_mask=True, block_rows=256,
                                block_cols=BC)
    h2, f2, hs2 = _proj(h_enc, W_d0, a_src_d0, block_rows=br)
    h2p = jnp.pad(h2, ((0, n_pad - N), (0, 0)))
    x_hat = _flash_layer(f2, h2p, a_dst_d0, hs2, mask8,
                         emit_mask=False, block_rows=256, block_cols=BC)
    return (h_enc, x_hat)
